# baseline (speedup 1.0000x reference)
import functools

import jax
import jax.numpy as jnp
from jax import lax
from jax.experimental import pallas as pl
from jax.experimental.pallas import tpu as pltpu
from jax.experimental.pallas import tpu_sc as plsc

_B = 1024
_N = 100000
_D = 16
_K = 3
_NUM_LABELS = 10
_PRED_TOL = 0.25

_NT = 2048
_T = (_N + _NT - 1) // _NT
_NPAD = _T * _NT

_NEG_INF = float("-inf")
_BIG_I = 2**30


_RB = 64
_NCH = _NT // 128


def _topk_body(x2_ref, t_ref, f_ref, rv1, rv2, rv3, ri1, ri2, ri3):
    nt = pl.program_id(0)

    @pl.when(nt == 0)
    def _init():
        for r in (rv1, rv2, rv3):
            r[...] = jnp.full((_B, 128), _NEG_INF, jnp.float32)
        for r in (ri1, ri2, ri3):
            r[...] = jnp.full((_B, 128), _BIG_I, jnp.int32)

    x2 = x2_ref[...]
    t = t_ref[...]
    f = f_ref[...]
    tw = t * f
    st = jnp.sum(tw * tw, axis=1)
    sx = 0.25 * jnp.sum(x2 * x2, axis=1)
    cross2 = lax.dot_general(
        x2, tw, (((1,), (1,)), ((), ())),
        preferred_element_type=jnp.float32)
    col = nt * _NT + lax.iota(jnp.int32, _NT)
    st = jnp.where(col < _N, st, jnp.inf)

    lane = lax.broadcasted_iota(jnp.int32, (1, 128), 1)
    for rb in range(_B // _RB):
        rs = pl.ds(rb * _RB, _RB)
        sx_b = sx[rb * _RB:(rb + 1) * _RB, None]
        v1, v2, v3 = rv1[rs, :], rv2[rs, :], rv3[rs, :]
        i1, i2, i3 = ri1[rs, :], ri2[rs, :], ri3[rs, :]
        for ch in range(_NCH):
            c2 = cross2[rb * _RB:(rb + 1) * _RB, ch * 128:(ch + 1) * 128]
            st_c = st[None, ch * 128:(ch + 1) * 128]
            q = sx_b + st_c
            v = c2 - q * lax.rsqrt(q)
            g = (nt * _NT + ch * 128) + lane
            gb = jnp.broadcast_to(g, (_RB, 128))
            c1 = v > v1
            c2_ = v > v2
            c3 = v > v3
            a2 = jnp.where(c1, v1, v)
            b2 = jnp.where(c1, i1, gb)
            a3 = jnp.where(c2_, v2, v)
            b3 = jnp.where(c2_, i2, gb)
            v1 = jnp.where(c1, v, v1)
            i1 = jnp.where(c1, gb, i1)
            v2 = jnp.where(c2_, a2, v2)
            i2 = jnp.where(c2_, b2, i2)
            v3 = jnp.where(c3, a3, v3)
            i3 = jnp.where(c3, b3, i3)
        rv1[rs, :], rv2[rs, :], rv3[rs, :] = v1, v2, v3
        ri1[rs, :], ri2[rs, :], ri3[rs, :] = i1, i2, i3


def _fin_body(rv1, rv2, rv3, ri1, ri2, ri3, vals_ref, idx_ref):
    cv = jnp.concatenate([rv1[...], rv2[...], rv3[...]], axis=1)
    ci = jnp.concatenate([ri1[...], ri2[...], ri3[...]], axis=1)
    bv, bi = [], []
    for k in range(_K):
        m = jnp.max(cv, axis=1, keepdims=True)
        p = jnp.min(jnp.where(cv == m, ci, _BIG_I), axis=1, keepdims=True)
        bv.append(m)
        bi.append(p)
        if k < _K - 1:
            cv = jnp.where(ci == p, _NEG_INF, cv)
    i3_ = lax.broadcasted_iota(jnp.int32, (_B, _K), 1)
    vals_ref[...] = jnp.where(i3_ == 0, bv[0],
                              jnp.where(i3_ == 1, bv[1], bv[2]))
    idx_ref[...] = jnp.where(i3_ == 0, bi[0],
                             jnp.where(i3_ == 1, bi[1], bi[2]))


@functools.partial(jax.jit, static_argnames=("interpret",))
def _tc_topk(x, train, features, interpret=False):
    t_pad = jnp.pad(train, ((0, _NPAD - _N), (0, 0)))
    f2 = features.reshape(1, _D)
    x2 = x * 2.0
    return pl.pallas_call(
        _topk_body,
        grid=(_T,),
        in_specs=[
            pl.BlockSpec((_B, _D), lambda n: (0, 0)),
            pl.BlockSpec((_NT, _D), lambda n: (n, 0)),
            pl.BlockSpec((1, _D), lambda n: (0, 0)),
        ],
        out_specs=[pl.BlockSpec((_B, 128), lambda n: (0, 0))] * 6,
        out_shape=[jax.ShapeDtypeStruct((_B, 128), jnp.float32)] * 3
        + [jax.ShapeDtypeStruct((_B, 128), jnp.int32)] * 3,
        compiler_params=pltpu.CompilerParams(
            dimension_semantics=("arbitrary",)),
        interpret=interpret,
    )(x2, t_pad, f2)


@functools.partial(jax.jit, static_argnames=("interpret",))
def _tc_fin(state, interpret=False):
    return pl.pallas_call(
        _fin_body,
        out_shape=[
            jax.ShapeDtypeStruct((_B, _K), jnp.float32),
            jax.ShapeDtypeStruct((_B, _K), jnp.int32),
        ],
        interpret=interpret,
    )(*state)


def _make_sc_post():
    nw = 32
    rpw = _B // nw
    opw = rpw * _NUM_LABELS
    mesh = plsc.VectorSubcoreMesh(core_axis_name="c", subcore_axis_name="s")

    @functools.partial(
        pl.kernel,
        out_type=[
            jax.ShapeDtypeStruct((_K * _B,), jnp.int32),
            jax.ShapeDtypeStruct((nw, _NUM_LABELS, rpw), jnp.float32),
            jax.ShapeDtypeStruct((nw, 16), jnp.int32),
        ],
        mesh=mesh,
        scratch_types=[
            pltpu.VMEM((rpw,), jnp.int32),
            pltpu.VMEM((rpw,), jnp.int32),
            pltpu.VMEM((rpw,), jnp.int32),
            pltpu.VMEM((rpw,), jnp.int32),
            pltpu.VMEM((rpw,), jnp.int32),
            pltpu.VMEM((_NUM_LABELS, rpw), jnp.float32),
            pltpu.VMEM((16,), jnp.int32),
            pltpu.SemaphoreType.DMA,
        ],
    )
    def sc_post(idxt_hbm, tl_hbm, y_hbm, lab_out, oh_out, cnt_out,
                idx_v, l0_v, l1_v, l2_v, y_v, oh_v, cnt_v, sem):
        wid = lax.axis_index("s") * 2 + lax.axis_index("c")
        base = wid * rpw
        for c, lv in enumerate((l0_v, l1_v, l2_v)):
            pltpu.sync_copy(idxt_hbm.at[pl.ds(c * _B + base, rpw)], idx_v)
            pltpu.async_copy(tl_hbm.at[idx_v], lv, sem).wait()
            pltpu.sync_copy(lv, lab_out.at[pl.ds(c * _B + base, rpw)])
        pltpu.sync_copy(y_hbm.at[pl.ds(base, rpw)], y_v)
        cnt = jnp.zeros((16,), jnp.int32)
        for g in range(rpw // 16):
            sl = pl.ds(g * 16, 16)
            out = lax.div(l0_v[sl] + l1_v[sl] + l2_v[sl], jnp.int32(_K))
            for c in range(_NUM_LABELS):
                oh_v[c, pl.ds(g * 16, 16)] = jnp.where(
                    out == c, jnp.float32(1.0), jnp.float32(0.0))
            y = y_v[sl]
            mag = (jnp.abs(y - out).astype(jnp.float32)
                   / (y + 1).astype(jnp.float32))
            cnt = cnt + jnp.where(mag < _PRED_TOL, 1, 0)
        pltpu.sync_copy(oh_v, oh_out.at[wid])
        cnt_v[...] = cnt
        pltpu.sync_copy(cnt_v, cnt_out.at[wid])

    return sc_post


_SC_POST_CACHE = []


def _sc_post(idx_flat, train_labels, y_labels):
    if not _SC_POST_CACHE:
        _SC_POST_CACHE.append(_make_sc_post())
    return _SC_POST_CACHE[0](idx_flat, train_labels, y_labels)


def kernel(x_input, train_inputs, train_labels, y_labels, features):
    state = _tc_topk(x_input, train_inputs, features)
    values, indices = _tc_fin(state)
    idx_t = indices.T.reshape(-1)
    lab_t, oh_t, cnt = _sc_post(idx_t, train_labels, y_labels)
    labels = lab_t.reshape(_K, _B).T
    one_hot_out = oh_t.transpose(0, 2, 1).reshape(_B, _NUM_LABELS)
    pred = jnp.sum(cnt).astype(jnp.float32) / jnp.float32(_B)
    return values, indices, labels, one_hot_out, pred


# speedup vs baseline: 1.0057x; 1.0057x over previous
import functools

import jax
import jax.numpy as jnp
from jax import lax
from jax.experimental import pallas as pl
from jax.experimental.pallas import tpu as pltpu
from jax.experimental.pallas import tpu_sc as plsc

_B = 1024
_N = 100000
_D = 16
_K = 3
_NUM_LABELS = 10
_PRED_TOL = 0.25

_NT = 5120
_T = (_N + _NT - 1) // _NT
_NPAD = _T * _NT

_NEG_INF = float("-inf")
_BIG_I = 2**30


_RB = 64
_NCH = _NT // 128


def _topk_body(x2_ref, t_ref, f_ref, vals_ref, idx_ref,
               rv1, rv2, rv3, ri1, ri2, ri3):
    nt = pl.program_id(0)

    @pl.when(nt == 0)
    def _init():
        for r in (rv1, rv2, rv3):
            r[...] = jnp.full((_B, 128), _NEG_INF, jnp.float32)
        for r in (ri1, ri2, ri3):
            r[...] = jnp.full((_B, 128), _BIG_I, jnp.int32)

    x2 = x2_ref[...]
    t = t_ref[...]
    f = f_ref[...]
    tw = t * f
    st = jnp.sum(tw * tw, axis=1)
    sx = 0.25 * jnp.sum(x2 * x2, axis=1)
    cross2 = lax.dot_general(
        x2, tw, (((1,), (1,)), ((), ())),
        preferred_element_type=jnp.float32)
    col = nt * _NT + lax.iota(jnp.int32, _NT)
    st = jnp.where(col < _N, st, jnp.inf)

    lane = lax.broadcasted_iota(jnp.int32, (1, 128), 1)
    for rb in range(_B // _RB):
        rs = pl.ds(rb * _RB, _RB)
        sx_b = sx[rb * _RB:(rb + 1) * _RB, None]
        v1, v2, v3 = rv1[rs, :], rv2[rs, :], rv3[rs, :]
        i1, i2, i3 = ri1[rs, :], ri2[rs, :], ri3[rs, :]
        for ch in range(_NCH):
            c2 = cross2[rb * _RB:(rb + 1) * _RB, ch * 128:(ch + 1) * 128]
            st_c = st[None, ch * 128:(ch + 1) * 128]
            q = sx_b + st_c
            v = c2 - q * lax.rsqrt(q)
            g = (nt * _NT + ch * 128) + lane
            gb = jnp.broadcast_to(g, (_RB, 128))
            c1 = v > v1
            c2_ = v > v2
            c3 = v > v3
            a2 = jnp.where(c1, v1, v)
            b2 = jnp.where(c1, i1, gb)
            a3 = jnp.where(c2_, v2, v)
            b3 = jnp.where(c2_, i2, gb)
            v1 = jnp.where(c1, v, v1)
            i1 = jnp.where(c1, gb, i1)
            v2 = jnp.where(c2_, a2, v2)
            i2 = jnp.where(c2_, b2, i2)
            v3 = jnp.where(c3, a3, v3)
            i3 = jnp.where(c3, b3, i3)
        rv1[rs, :], rv2[rs, :], rv3[rs, :] = v1, v2, v3
        ri1[rs, :], ri2[rs, :], ri3[rs, :] = i1, i2, i3

    @pl.when(nt == _T - 1)
    def _fin():
        cv = jnp.concatenate([rv1[...], rv2[...], rv3[...]], axis=1)
        ci = jnp.concatenate([ri1[...], ri2[...], ri3[...]], axis=1)
        bv, bi = [], []
        for k in range(_K):
            m = jnp.max(cv, axis=1, keepdims=True)
            p = jnp.min(jnp.where(cv == m, ci, _BIG_I), axis=1,
                        keepdims=True)
            bv.append(m)
            bi.append(p)
            if k < _K - 1:
                cv = jnp.where(ci == p, _NEG_INF, cv)
        i3_ = lax.broadcasted_iota(jnp.int32, (_B, _K), 1)
        vals_ref[...] = jnp.where(i3_ == 0, bv[0],
                                  jnp.where(i3_ == 1, bv[1], bv[2]))
        idx_ref[...] = jnp.where(i3_ == 0, bi[0],
                                 jnp.where(i3_ == 1, bi[1], bi[2]))


@functools.partial(jax.jit, static_argnames=("interpret",))
def _tc_topk(x, train, features, interpret=False):
    t_pad = jnp.pad(train, ((0, _NPAD - _N), (0, 0)))
    f2 = features.reshape(1, _D)
    x2 = x * 2.0
    return pl.pallas_call(
        _topk_body,
        grid=(_T,),
        in_specs=[
            pl.BlockSpec((_B, _D), lambda n: (0, 0)),
            pl.BlockSpec((_NT, _D), lambda n: (n, 0)),
            pl.BlockSpec((1, _D), lambda n: (0, 0)),
        ],
        out_specs=[
            pl.BlockSpec((_B, _K), lambda n: (0, 0)),
            pl.BlockSpec((_B, _K), lambda n: (0, 0)),
        ],
        out_shape=[
            jax.ShapeDtypeStruct((_B, _K), jnp.float32),
            jax.ShapeDtypeStruct((_B, _K), jnp.int32),
        ],
        scratch_shapes=[
            pltpu.VMEM((_B, 128), jnp.float32),
            pltpu.VMEM((_B, 128), jnp.float32),
            pltpu.VMEM((_B, 128), jnp.float32),
            pltpu.VMEM((_B, 128), jnp.int32),
            pltpu.VMEM((_B, 128), jnp.int32),
            pltpu.VMEM((_B, 128), jnp.int32),
        ],
        compiler_params=pltpu.CompilerParams(
            dimension_semantics=("arbitrary",)),
        interpret=interpret,
    )(x2, t_pad, f2)


def _make_sc_post():
    nw = 32
    rpw = _B // nw
    opw = rpw * _NUM_LABELS
    mesh = plsc.VectorSubcoreMesh(core_axis_name="c", subcore_axis_name="s")

    @functools.partial(
        pl.kernel,
        out_type=[
            jax.ShapeDtypeStruct((_K * _B,), jnp.int32),
            jax.ShapeDtypeStruct((nw, _NUM_LABELS, rpw), jnp.float32),
            jax.ShapeDtypeStruct((nw, 16), jnp.int32),
        ],
        mesh=mesh,
        scratch_types=[
            pltpu.VMEM((rpw,), jnp.int32),
            pltpu.VMEM((rpw,), jnp.int32),
            pltpu.VMEM((rpw,), jnp.int32),
            pltpu.VMEM((rpw,), jnp.int32),
            pltpu.VMEM((rpw,), jnp.int32),
            pltpu.VMEM((_NUM_LABELS, rpw), jnp.float32),
            pltpu.VMEM((16,), jnp.int32),
            pltpu.SemaphoreType.DMA,
        ],
    )
    def sc_post(idxt_hbm, tl_hbm, y_hbm, lab_out, oh_out, cnt_out,
                idx_v, l0_v, l1_v, l2_v, y_v, oh_v, cnt_v, sem):
        wid = lax.axis_index("s") * 2 + lax.axis_index("c")
        base = wid * rpw
        for c, lv in enumerate((l0_v, l1_v, l2_v)):
            pltpu.sync_copy(idxt_hbm.at[pl.ds(c * _B + base, rpw)], idx_v)
            pltpu.async_copy(tl_hbm.at[idx_v], lv, sem).wait()
            pltpu.sync_copy(lv, lab_out.at[pl.ds(c * _B + base, rpw)])
        pltpu.sync_copy(y_hbm.at[pl.ds(base, rpw)], y_v)
        cnt = jnp.zeros((16,), jnp.int32)
        for g in range(rpw // 16):
            sl = pl.ds(g * 16, 16)
            out = lax.div(l0_v[sl] + l1_v[sl] + l2_v[sl], jnp.int32(_K))
            for c in range(_NUM_LABELS):
                oh_v[c, pl.ds(g * 16, 16)] = jnp.where(
                    out == c, jnp.float32(1.0), jnp.float32(0.0))
            y = y_v[sl]
            mag = (jnp.abs(y - out).astype(jnp.float32)
                   / (y + 1).astype(jnp.float32))
            cnt = cnt + jnp.where(mag < _PRED_TOL, 1, 0)
        pltpu.sync_copy(oh_v, oh_out.at[wid])
        cnt_v[...] = cnt
        pltpu.sync_copy(cnt_v, cnt_out.at[wid])

    return sc_post


_SC_POST_CACHE = []


def _sc_post(idx_flat, train_labels, y_labels):
    if not _SC_POST_CACHE:
        _SC_POST_CACHE.append(_make_sc_post())
    return _SC_POST_CACHE[0](idx_flat, train_labels, y_labels)


def kernel(x_input, train_inputs, train_labels, y_labels, features):
    values, indices = _tc_topk(x_input, train_inputs, features)
    idx_t = indices.T.reshape(-1)
    lab_t, oh_t, cnt = _sc_post(idx_t, train_labels, y_labels)
    labels = lab_t.reshape(_K, _B).T
    one_hot_out = oh_t.transpose(0, 2, 1).reshape(_B, _NUM_LABELS)
    pred = jnp.sum(cnt).astype(jnp.float32) / jnp.float32(_B)
    return values, indices, labels, one_hot_out, pred


# speedup vs baseline: 1.0152x; 1.0094x over previous
import functools

import jax
import jax.numpy as jnp
from jax import lax
from jax.experimental import pallas as pl
from jax.experimental.pallas import tpu as pltpu
from jax.experimental.pallas import tpu_sc as plsc

_B = 1024
_N = 100000
_D = 16
_K = 3
_NUM_LABELS = 10
_PRED_TOL = 0.25

_NT = 2048
_T = (_N + _NT - 1) // _NT
_NPAD = _T * _NT

_NEG_INF = float("-inf")
_BIG_I = 2**30


_RB = 64
_NCH = _NT // 128


def _topk_body(x2_ref, t_ref, f_ref, vals_ref, idx_ref,
               rv1, rv2, rv3, ri1, ri2, ri3):
    nt = pl.program_id(0)

    @pl.when(nt == 0)
    def _init():
        for r in (rv1, rv2, rv3):
            r[...] = jnp.full((_B, 128), _NEG_INF, jnp.float32)
        for r in (ri1, ri2, ri3):
            r[...] = jnp.full((_B, 128), _BIG_I, jnp.int32)

    x2 = x2_ref[...]
    t = t_ref[...]
    f = f_ref[...]
    tw = t * f
    st = jnp.sum(tw * tw, axis=1)
    sx = 0.25 * jnp.sum(x2 * x2, axis=1)
    cross2 = lax.dot_general(
        x2, tw, (((1,), (1,)), ((), ())),
        preferred_element_type=jnp.float32)
    col = nt * _NT + lax.iota(jnp.int32, _NT)
    st = jnp.where(col < _N, st, jnp.inf)

    lane = lax.broadcasted_iota(jnp.int32, (1, 128), 1)
    for rb in range(_B // _RB):
        rs = pl.ds(rb * _RB, _RB)
        sx_b = sx[rb * _RB:(rb + 1) * _RB, None]
        v1, v2, v3 = rv1[rs, :], rv2[rs, :], rv3[rs, :]
        i1, i2, i3 = ri1[rs, :], ri2[rs, :], ri3[rs, :]
        for ch in range(_NCH):
            c2 = cross2[rb * _RB:(rb + 1) * _RB, ch * 128:(ch + 1) * 128]
            st_c = st[None, ch * 128:(ch + 1) * 128]
            q = sx_b + st_c
            v = c2 - q * lax.rsqrt(q)
            g = (nt * _NT + ch * 128) + lane
            gb = jnp.broadcast_to(g, (_RB, 128))
            c1 = v > v1
            c2_ = v > v2
            c3 = v > v3
            a2 = jnp.where(c1, v1, v)
            b2 = jnp.where(c1, i1, gb)
            a3 = jnp.where(c2_, v2, v)
            b3 = jnp.where(c2_, i2, gb)
            v1 = jnp.where(c1, v, v1)
            i1 = jnp.where(c1, gb, i1)
            v2 = jnp.where(c2_, a2, v2)
            i2 = jnp.where(c2_, b2, i2)
            v3 = jnp.where(c3, a3, v3)
            i3 = jnp.where(c3, b3, i3)
        rv1[rs, :], rv2[rs, :], rv3[rs, :] = v1, v2, v3
        ri1[rs, :], ri2[rs, :], ri3[rs, :] = i1, i2, i3

    @pl.when(nt == _T - 1)
    def _fin():
        cv = jnp.concatenate([rv1[...], rv2[...], rv3[...]], axis=1)
        ci = jnp.concatenate([ri1[...], ri2[...], ri3[...]], axis=1)
        bv, bi = [], []
        for k in range(_K):
            m = jnp.max(cv, axis=1, keepdims=True)
            p = jnp.min(jnp.where(cv == m, ci, _BIG_I), axis=1,
                        keepdims=True)
            bv.append(m)
            bi.append(p)
            if k < _K - 1:
                cv = jnp.where(ci == p, _NEG_INF, cv)
        i3_ = lax.broadcasted_iota(jnp.int32, (_B, _K), 1)
        vals_ref[...] = jnp.where(i3_ == 0, bv[0],
                                  jnp.where(i3_ == 1, bv[1], bv[2]))
        idx_ref[...] = jnp.where(i3_ == 0, bi[0],
                                 jnp.where(i3_ == 1, bi[1], bi[2]))


@functools.partial(jax.jit, static_argnames=("interpret",))
def _tc_topk(x, train, features, interpret=False):
    t_pad = jnp.pad(train, ((0, _NPAD - _N), (0, 0)))
    f2 = features.reshape(1, _D)
    x2 = x * 2.0
    return pl.pallas_call(
        _topk_body,
        grid=(_T,),
        in_specs=[
            pl.BlockSpec((_B, _D), lambda n: (0, 0)),
            pl.BlockSpec((_NT, _D), lambda n: (n, 0)),
            pl.BlockSpec((1, _D), lambda n: (0, 0)),
        ],
        out_specs=[
            pl.BlockSpec((_B, _K), lambda n: (0, 0)),
            pl.BlockSpec((_B, _K), lambda n: (0, 0)),
        ],
        out_shape=[
            jax.ShapeDtypeStruct((_B, _K), jnp.float32),
            jax.ShapeDtypeStruct((_B, _K), jnp.int32),
        ],
        scratch_shapes=[
            pltpu.VMEM((_B, 128), jnp.float32),
            pltpu.VMEM((_B, 128), jnp.float32),
            pltpu.VMEM((_B, 128), jnp.float32),
            pltpu.VMEM((_B, 128), jnp.int32),
            pltpu.VMEM((_B, 128), jnp.int32),
            pltpu.VMEM((_B, 128), jnp.int32),
        ],
        compiler_params=pltpu.CompilerParams(
            dimension_semantics=("arbitrary",)),
        interpret=interpret,
    )(x2, t_pad, f2)


def _make_sc_post():
    nw = 32
    rpw = _B // nw
    opw = rpw * _NUM_LABELS
    mesh = plsc.VectorSubcoreMesh(core_axis_name="c", subcore_axis_name="s")

    ipw = rpw * _K

    def _dg(vec, pattern):
        dnums = lax.GatherDimensionNumbers(
            offset_dims=(), collapsed_slice_dims=(0,), start_index_map=(0,))
        return lax.gather(
            vec, pattern[:, None], dnums, slice_sizes=(1,),
            mode=lax.GatherScatterMode.PROMISE_IN_BOUNDS)

    @functools.partial(
        pl.kernel,
        out_type=[
            jax.ShapeDtypeStruct((_B * _K,), jnp.int32),
            jax.ShapeDtypeStruct((_B * _NUM_LABELS,), jnp.float32),
            jax.ShapeDtypeStruct((nw, 16), jnp.int32),
        ],
        mesh=mesh,
        scratch_types=[
            pltpu.VMEM((ipw,), jnp.int32),
            pltpu.VMEM((ipw,), jnp.int32),
            pltpu.VMEM((rpw,), jnp.int32),
            pltpu.VMEM((opw,), jnp.float32),
            pltpu.VMEM((16,), jnp.int32),
            pltpu.SemaphoreType.DMA,
        ],
    )
    def sc_post(idx_hbm, tl_hbm, y_hbm, lab_out, oh_out, cnt_out,
                idx_v, lab_v, y_v, oh_v, cnt_v, sem):
        wid = lax.axis_index("s") * 2 + lax.axis_index("c")
        base = wid * rpw
        pltpu.sync_copy(idx_hbm.at[pl.ds(wid * ipw, ipw)], idx_v)
        pltpu.async_copy(tl_hbm.at[idx_v], lab_v, sem).wait()
        pltpu.sync_copy(lab_v, lab_out.at[pl.ds(wid * ipw, ipw)])
        pltpu.sync_copy(y_hbm.at[pl.ds(base, rpw)], y_v)
        iota = lax.iota(jnp.int32, 16)
        cnt = jnp.zeros((16,), jnp.int32)
        for g in range(rpw // 16):
            regs = [lab_v[pl.ds(g * 48 + r * 16, 16)] for r in range(3)]
            comps = []
            for c in range(_K):
                pos = iota * _K + c
                lanep = lax.rem(pos, jnp.int32(16))
                regid = lax.div(pos, jnp.int32(16))
                comp = jnp.where(
                    regid == 0, _dg(regs[0], lanep),
                    jnp.where(regid == 1, _dg(regs[1], lanep),
                              _dg(regs[2], lanep)))
                comps.append(comp)
            out = lax.div(comps[0] + comps[1] + comps[2], jnp.int32(_K))
            for k in range(_NUM_LABELS):
                p = 16 * k + iota
                rowp = lax.div(p, jnp.int32(_NUM_LABELS))
                clsp = lax.rem(p, jnp.int32(_NUM_LABELS))
                ohk = jnp.where(_dg(out, rowp) == clsp,
                                jnp.float32(1.0), jnp.float32(0.0))
                oh_v[pl.ds(g * 160 + 16 * k, 16)] = ohk
            y = y_v[pl.ds(g * 16, 16)]
            mag = (jnp.abs(y - out).astype(jnp.float32)
                   / (y + 1).astype(jnp.float32))
            cnt = cnt + jnp.where(mag < _PRED_TOL, 1, 0)
        pltpu.sync_copy(oh_v, oh_out.at[pl.ds(wid * opw, opw)])
        cnt_v[...] = cnt
        pltpu.sync_copy(cnt_v, cnt_out.at[wid])

    return sc_post


_SC_POST_CACHE = []


def _sc_post(idx_flat, train_labels, y_labels):
    if not _SC_POST_CACHE:
        _SC_POST_CACHE.append(_make_sc_post())
    return _SC_POST_CACHE[0](idx_flat, train_labels, y_labels)


def kernel(x_input, train_inputs, train_labels, y_labels, features):
    values, indices = _tc_topk(x_input, train_inputs, features)
    lab_f, oh_f, cnt = _sc_post(indices.reshape(-1), train_labels, y_labels)
    labels = lab_f.reshape(_B, _K)
    one_hot_out = oh_f.reshape(_B, _NUM_LABELS)
    pred = jnp.sum(cnt).astype(jnp.float32) / jnp.float32(_B)
    return values, indices, labels, one_hot_out, pred
